# baseline (speedup 1.0000x reference)
import jax
import jax.numpy as jnp
from jax.experimental import pallas as pl
from jax.experimental.pallas import tpu as pltpu

BATCH = 8
WIDTH = 32
N_NODE = 2048
N_REL = 4


def _rowscale(col):
    return jnp.concatenate([col] * BATCH, axis=0)


def _lerp_kernel(db_ref, x_ref, w_ref, eq_ref, out_ref, acc_ref):
    r = pl.program_id(0)

    wsm = jax.nn.softmax(w_ref[...], axis=1)

    def sel(base):
        c = wsm[:, base + N_REL - 1 : base + N_REL]
        for k in range(N_REL - 2, -1, -1):
            c = jnp.where(r == k, wsm[:, base + k : base + k + 1], c)
        return c

    w1m = _rowscale(sel(0))
    w2m = _rowscale(sel(N_REL))

    d = db_ref[0]
    x = x_ref[...]
    xs1 = x * w1m
    xs2 = x * w2m

    y = jax.lax.dot_general(
        xs1, d, (((1,), (0,)), ((), ())), preferred_element_type=jnp.float32
    )
    y += jax.lax.dot_general(
        xs2, d, (((1,), (1,)), ((), ())), preferred_element_type=jnp.float32
    )

    @pl.when(r == 0)
    def _first():
        acc_ref[...] = y

    @pl.when(r > 0)
    def _rest():
        acc_ref[...] += y

    @pl.when(r == N_REL - 1)
    def _fin():
        eqsm = jax.nn.softmax(eq_ref[...], axis=1)
        eq0 = _rowscale(eqsm[:, 0:1])
        eq1 = _rowscale(eqsm[:, 1:2])
        acc = acc_ref[...]
        out_ref[...] = (1.0 - jnp.exp(-acc)) * eq0 + x * eq1


@jax.jit
def kernel(inputs, database, weights, equity_weight):
    m = BATCH * WIDTH
    x = inputs.reshape(m, N_NODE)
    out2d = pl.pallas_call(
        _lerp_kernel,
        grid=(N_REL,),
        in_specs=[
            pl.BlockSpec((1, N_NODE, N_NODE), lambda r: (r, 0, 0)),
            pl.BlockSpec((m, N_NODE), lambda r: (0, 0)),
            pl.BlockSpec((WIDTH, 2 * N_REL), lambda r: (0, 0)),
            pl.BlockSpec((WIDTH, 2), lambda r: (0, 0)),
        ],
        out_specs=pl.BlockSpec((m, N_NODE), lambda r: (0, 0)),
        out_shape=jax.ShapeDtypeStruct((m, N_NODE), jnp.float32),
        scratch_shapes=[pltpu.VMEM((m, N_NODE), jnp.float32)],
    )(database, x, weights, equity_weight)
    return out2d.reshape(BATCH, WIDTH, N_NODE)
